# baseline (speedup 1.0000x reference)
import functools

import jax
import jax.numpy as jnp
from jax.experimental import pallas as pl

_BLK = 2048
_OUT_ROWS = 16


def _router_body(x_ref, ut_ref, w1t_ref, b1_ref, w2t_ref, b2_ref,
                 tabt_ref, shp_ref, idx_ref):
    n_arcs = ut_ref.shape[0]
    arc_p = tabt_ref.shape[1]
    blk = x_ref.shape[0]
    f32 = jnp.float32
    ht = jnp.maximum(
        jax.lax.dot_general(w1t_ref[...], x_ref[...],
                            (((1,), (1,)), ((), ())),
                            preferred_element_type=f32) + b1_ref[...], 0.0)
    st = (jax.lax.dot_general(w2t_ref[...], ht,
                              (((1,), (0,)), ((), ())),
                              preferred_element_type=f32) + b2_ref[...])
    logits = jax.nn.sigmoid(st)
    u = jnp.clip(ut_ref[...], 1e-10, 1.0 - 1e-10)
    g = -jnp.log(-jnp.log(u))
    score = logits + g
    m = jnp.max(score, axis=0, keepdims=True)
    iota = jax.lax.broadcasted_iota(jnp.int32, score.shape, 0)
    idx = jnp.min(jnp.where(score == m, iota, n_arcs),
                  axis=0, keepdims=True)
    iota_p = jax.lax.broadcasted_iota(jnp.int32, (arc_p, blk), 0)
    one_hot = (iota_p == idx).astype(jnp.bfloat16)
    shp_ref[...] = jax.lax.dot_general(tabt_ref[...], one_hot,
                                       (((1,), (0,)), ((), ())),
                                       preferred_element_type=f32)
    idx_ref[...] = idx


@functools.partial(jax.jit, static_argnames=())
def kernel(state, gumbel_u, W1, b1, W2, b2, shape_table):
    tokens, obs_dim = state.shape
    hidden = W1.shape[1]
    n_arcs = W2.shape[1]
    tab_w = shape_table.shape[1]
    arc_p = (n_arcs + 127) // 128 * 128
    ut = gumbel_u.T
    w1t = W1.T
    w2t = W2.T
    tabt = jnp.pad(shape_table.T.astype(jnp.bfloat16),
                   ((0, 0), (0, arc_p - n_arcs)))
    b1c = b1.reshape(hidden, 1)
    b2c = b2.reshape(n_arcs, 1)
    grid = (tokens // _BLK,)
    out = pl.pallas_call(
        _router_body,
        grid=grid,
        in_specs=[
            pl.BlockSpec((_BLK, obs_dim), lambda i: (i, 0)),
            pl.BlockSpec((n_arcs, _BLK), lambda i: (0, i)),
            pl.BlockSpec((hidden, obs_dim), lambda i: (0, 0)),
            pl.BlockSpec((hidden, 1), lambda i: (0, 0)),
            pl.BlockSpec((n_arcs, hidden), lambda i: (0, 0)),
            pl.BlockSpec((n_arcs, 1), lambda i: (0, 0)),
            pl.BlockSpec((tab_w, arc_p), lambda i: (0, 0)),
        ],
        out_specs=[
            pl.BlockSpec((tab_w, _BLK), lambda i: (0, i)),
            pl.BlockSpec((1, _BLK), lambda i: (0, i)),
        ],
        out_shape=[
            jax.ShapeDtypeStruct((tab_w, tokens), jnp.float32),
            jax.ShapeDtypeStruct((1, tokens), jnp.int32),
        ],
    )(state, ut, w1t, b1c, w2t, b2c, tabt)
    shp, idx = out
    return shp.T, idx.reshape(tokens)


# speedup vs baseline: 1.0099x; 1.0099x over previous
import functools

import jax
import jax.numpy as jnp
from jax.experimental import pallas as pl

_BLK = 1024
_OUT_ROWS = 16


def _router_body(x_ref, ut_ref, w1t_ref, b1_ref, w2t_ref, b2_ref,
                 tabt_ref, shp_ref, idx_ref):
    n_arcs = ut_ref.shape[0]
    arc_p = tabt_ref.shape[1]
    blk = x_ref.shape[0]
    f32 = jnp.float32
    ht = jnp.maximum(
        jax.lax.dot_general(w1t_ref[...], x_ref[...],
                            (((1,), (1,)), ((), ())),
                            preferred_element_type=f32) + b1_ref[...], 0.0)
    st = (jax.lax.dot_general(w2t_ref[...], ht,
                              (((1,), (0,)), ((), ())),
                              preferred_element_type=f32) + b2_ref[...])
    logits = jax.nn.sigmoid(st)
    u = jnp.clip(ut_ref[...], 1e-10, 1.0 - 1e-10)
    g = -jnp.log(-jnp.log(u))
    score = logits + g
    m = jnp.max(score, axis=0, keepdims=True)
    iota = jax.lax.broadcasted_iota(jnp.int32, score.shape, 0)
    idx = jnp.min(jnp.where(score == m, iota, n_arcs),
                  axis=0, keepdims=True)
    iota_p = jax.lax.broadcasted_iota(jnp.int32, (arc_p, blk), 0)
    one_hot = (iota_p == idx).astype(jnp.bfloat16)
    shp_ref[...] = jax.lax.dot_general(tabt_ref[...], one_hot,
                                       (((1,), (0,)), ((), ())),
                                       preferred_element_type=f32)
    idx_ref[...] = idx


@functools.partial(jax.jit, static_argnames=())
def kernel(state, gumbel_u, W1, b1, W2, b2, shape_table):
    tokens, obs_dim = state.shape
    hidden = W1.shape[1]
    n_arcs = W2.shape[1]
    tab_w = shape_table.shape[1]
    arc_p = (n_arcs + 127) // 128 * 128
    ut = gumbel_u.T
    w1t = W1.T
    w2t = W2.T
    tabt = jnp.pad(shape_table.T.astype(jnp.bfloat16),
                   ((0, 0), (0, arc_p - n_arcs)))
    b1c = b1.reshape(hidden, 1)
    b2c = b2.reshape(n_arcs, 1)
    grid = (tokens // _BLK,)
    out = pl.pallas_call(
        _router_body,
        grid=grid,
        in_specs=[
            pl.BlockSpec((_BLK, obs_dim), lambda i: (i, 0)),
            pl.BlockSpec((n_arcs, _BLK), lambda i: (0, i)),
            pl.BlockSpec((hidden, obs_dim), lambda i: (0, 0)),
            pl.BlockSpec((hidden, 1), lambda i: (0, 0)),
            pl.BlockSpec((n_arcs, hidden), lambda i: (0, 0)),
            pl.BlockSpec((n_arcs, 1), lambda i: (0, 0)),
            pl.BlockSpec((tab_w, arc_p), lambda i: (0, 0)),
        ],
        out_specs=[
            pl.BlockSpec((tab_w, _BLK), lambda i: (0, i)),
            pl.BlockSpec((1, _BLK), lambda i: (0, i)),
        ],
        out_shape=[
            jax.ShapeDtypeStruct((tab_w, tokens), jnp.float32),
            jax.ShapeDtypeStruct((1, tokens), jnp.int32),
        ],
    )(state, ut, w1t, b1c, w2t, b2c, tabt)
    shp, idx = out
    return shp.T, idx.reshape(tokens)


# speedup vs baseline: 1.0738x; 1.0633x over previous
import functools

import jax
import jax.numpy as jnp
from jax.experimental import pallas as pl

_BLK = 1024
_OUT_ROWS = 16


def _router_body(x_ref, ut_ref, w1t_ref, b1_ref, w2_ref, b2_ref,
                 tabt_ref, shp_ref, idx_ref):
    n_arcs = ut_ref.shape[0]
    arc_p = tabt_ref.shape[1]
    blk = x_ref.shape[0]
    f32 = jnp.float32
    ht = jnp.maximum(
        jax.lax.dot_general(w1t_ref[...], x_ref[...],
                            (((1,), (1,)), ((), ())),
                            preferred_element_type=f32) + b1_ref[...], 0.0)
    st = (jax.lax.dot_general(w2_ref[...], ht,
                              (((0,), (0,)), ((), ())),
                              preferred_element_type=f32) + b2_ref[...])
    logits = jax.nn.sigmoid(st)
    u = jnp.clip(ut_ref[...], 1e-10, 1.0 - 1e-10)
    g = -jnp.log(-jnp.log(u))
    score = logits + g
    m = jnp.max(score, axis=0, keepdims=True)
    iota = jax.lax.broadcasted_iota(jnp.int32, score.shape, 0)
    idx = jnp.min(jnp.where(score == m, iota, n_arcs),
                  axis=0, keepdims=True)
    iota_p = jax.lax.broadcasted_iota(jnp.int32, (arc_p, blk), 0)
    one_hot = (iota_p == idx).astype(jnp.bfloat16)
    shp_ref[...] = jax.lax.dot_general(tabt_ref[...], one_hot,
                                       (((1,), (0,)), ((), ())),
                                       preferred_element_type=f32)
    idx_ref[...] = idx


@functools.partial(jax.jit, static_argnames=())
def kernel(state, gumbel_u, W1, b1, W2, b2, shape_table):
    tokens, obs_dim = state.shape
    hidden = W1.shape[1]
    n_arcs = W2.shape[1]
    tab_w = shape_table.shape[1]
    arc_p = (n_arcs + 127) // 128 * 128
    ut = gumbel_u.T
    w1t = W1.T
    tabt = jnp.pad(shape_table.T.astype(jnp.bfloat16),
                   ((0, 0), (0, arc_p - n_arcs)))
    b1c = b1.reshape(hidden, 1)
    b2c = b2.reshape(n_arcs, 1)
    grid = (tokens // _BLK,)
    out = pl.pallas_call(
        _router_body,
        grid=grid,
        in_specs=[
            pl.BlockSpec((_BLK, obs_dim), lambda i: (i, 0)),
            pl.BlockSpec((n_arcs, _BLK), lambda i: (0, i)),
            pl.BlockSpec((hidden, obs_dim), lambda i: (0, 0)),
            pl.BlockSpec((hidden, 1), lambda i: (0, 0)),
            pl.BlockSpec((hidden, n_arcs), lambda i: (0, 0)),
            pl.BlockSpec((n_arcs, 1), lambda i: (0, 0)),
            pl.BlockSpec((tab_w, arc_p), lambda i: (0, 0)),
        ],
        out_specs=[
            pl.BlockSpec((tab_w, _BLK), lambda i: (0, i)),
            pl.BlockSpec((1, _BLK), lambda i: (0, i)),
        ],
        out_shape=[
            jax.ShapeDtypeStruct((tab_w, tokens), jnp.float32),
            jax.ShapeDtypeStruct((1, tokens), jnp.int32),
        ],
    )(state, ut, w1t, b1c, W2, b2c, tabt)
    shp, idx = out
    return shp.T, idx.reshape(tokens)
